# baseline (speedup 1.0000x reference)
import jax
import jax.numpy as jnp
from jax import lax
from jax.experimental import pallas as pl
from jax.experimental.pallas import tpu as pltpu
from jax.experimental.pallas import tpu_sc as plsc

NI = 4096
NJ = 200
D = 10
VOC = 100
VP = 112
NC, NS, L = 2, 16, 16
NW = NC * NS
IB = NI // NW
JB = 40
NB = NJ // JB
M = IB // L


X_BLK_BYTES = JB * IB * 4
OUT_HALF_BYTES = D * (JB // 2) * IB * 4
TBL_BYTES = D * VP * 4


def _sc_body(x_hbm, tbl_hbm, out_hbm, x_v, tbl_raw_v, tbl_v, out_v,
             sem_t, sem_x, sem_o):
    wid = lax.axis_index("s") * NC + lax.axis_index("c")
    i0 = wid * IB
    def start_x(b, p):
        return pltpu.async_copy(
            x_hbm.at[pl.ds(b * JB, JB), pl.ds(i0, IB)], x_v.at[p], sem_x)

    tbl_copy = pltpu.async_copy(tbl_hbm, tbl_raw_v, sem_t)
    x_copies = [start_x(0, 0)]
    out_copies = []

    tbl_copy.wait()
    lane = lax.iota(jnp.int32, L)
    for h in range(VP // L):
        src = (lane + h * L) * D
        msk = (lane + h * L) < VOC
        for d in range(D):
            v = plsc.load_gather(tbl_raw_v, [src + d], mask=msk)
            tbl_v[d, pl.ds(h * L, L)] = v

    for b in range(NB):
        p = b % 2
        if b + 1 < NB:
            x_copies.append(start_x(b + 1, (b + 1) % 2))
        x_copies[b].wait()
        if b >= 2:
            out_copies[b - 2].wait()

        @plsc.parallel_loop(0, JB, unroll=2)
        def _row(j):
            idx = [x_v[p, j, pl.ds(m * L, L)] for m in range(M)]
            for d in range(D):
                for m in range(M):
                    v = plsc.load_gather(tbl_v.at[d], [idx[m]])
                    out_v[p, d, j, pl.ds(m * L, L)] = v

        out_copies.append(pltpu.async_copy(
            out_v.at[p],
            out_hbm.at[:, pl.ds(b * JB, JB), pl.ds(i0, IB)],
            sem_o))

    for cp in out_copies[NB - 2:]:
        cp.wait()


@jax.jit
def _sc_lookup(x_t, tbl_t):
    mesh = plsc.VectorSubcoreMesh(core_axis_name="c", subcore_axis_name="s")
    f = pl.kernel(
        _sc_body,
        mesh=mesh,
        out_type=jax.ShapeDtypeStruct((D, NJ, NI), jnp.float32),
        scratch_types=[
            pltpu.VMEM((2, JB, IB), jnp.int32),
            pltpu.VMEM((VOC * D,), jnp.float32),
            pltpu.VMEM((D, VP), jnp.float32),
            pltpu.VMEM((2, D, JB, IB), jnp.float32),
            pltpu.SemaphoreType.DMA,
            pltpu.SemaphoreType.DMA,
            pltpu.SemaphoreType.DMA,
        ],
        compiler_params=pltpu.CompilerParams(needs_layout_passes=False),
    )
    return f(x_t, tbl_t)


def kernel(x, table):
    out_t = _sc_lookup(x.T, table.reshape(-1))
    return out_t.transpose(2, 1, 0)


# speedup vs baseline: 1.0152x; 1.0152x over previous
import jax
import jax.numpy as jnp
from jax import lax
from jax.experimental import pallas as pl
from jax.experimental.pallas import tpu as pltpu
from jax.experimental.pallas import tpu_sc as plsc

NI = 4096
NJ = 200
D = 10
VOC = 100
VP = 104
NC, NS, L = 2, 16, 16
NW = NC * NS
IB = NI // NW
JB = 40
NB = NJ // JB
M = IB // L


def _sc_body(x_hbm, tbl_hbm, out_hbm, x_v, tbl_v, out_v, sem_t, sem_x, sem_o):
    wid = lax.axis_index("s") * NC + lax.axis_index("c")
    i0 = wid * IB
    def start_x(b, p):
        return pltpu.async_copy(
            x_hbm.at[pl.ds(b * JB, JB), pl.ds(i0, IB)], x_v.at[p], sem_x)

    tbl_copy = pltpu.async_copy(tbl_hbm, tbl_v, sem_t)
    x_copies = [start_x(0, 0)]
    out_copies = []

    for b in range(NB):
        p = b % 2
        if b + 1 < NB:
            x_copies.append(start_x(b + 1, (b + 1) % 2))
        if b == 0:
            tbl_copy.wait()
        x_copies[b].wait()
        if b >= 2:
            out_copies[b - 2].wait()

        @plsc.parallel_loop(0, JB, unroll=2)
        def _row(j):
            idx = [x_v[p, j, pl.ds(m * L, L)] for m in range(M)]
            for d in range(D):
                for m in range(M):
                    v = plsc.load_gather(tbl_v.at[d], [idx[m]])
                    out_v[p, d, j, pl.ds(m * L, L)] = v

        out_copies.append(pltpu.async_copy(
            out_v.at[p],
            out_hbm.at[:, pl.ds(b * JB, JB), pl.ds(i0, IB)],
            sem_o))

    for cp in out_copies[NB - 2:]:
        cp.wait()


@jax.jit
def _sc_lookup(x_t, tbl_t):
    mesh = plsc.VectorSubcoreMesh(core_axis_name="c", subcore_axis_name="s")
    f = pl.kernel(
        _sc_body,
        mesh=mesh,
        out_type=jax.ShapeDtypeStruct((D, NJ, NI), jnp.float32),
        scratch_types=[
            pltpu.VMEM((2, JB, IB), jnp.int32),
            pltpu.VMEM((D, VP), jnp.float32),
            pltpu.VMEM((2, D, JB, IB), jnp.float32),
            pltpu.SemaphoreType.DMA,
            pltpu.SemaphoreType.DMA,
            pltpu.SemaphoreType.DMA,
        ],
        compiler_params=pltpu.CompilerParams(needs_layout_passes=False),
    )
    return f(x_t, tbl_t)


def kernel(x, table):
    tbl_t = jnp.zeros((D, VP), table.dtype).at[:, :VOC].set(table.T)
    out_t = _sc_lookup(x.T, tbl_t)
    return out_t.transpose(2, 1, 0)
